# baseline (speedup 1.0000x reference)
import functools

import jax
import jax.numpy as jnp
from jax import lax
from jax.experimental import pallas as pl
from jax.experimental.pallas import tpu as pltpu
from jax.experimental.pallas import tpu_sc as plsc

C = 24
BROW = 25
L = 16
NC = 2
NS = 16
NW = NC * NS
BCH = 256
SOCT = 8
NACC = 4


def _sc_score(ytp, ypt, bpad):
    n_seq, _, n_batch = ytp.shape
    grid = (n_batch // BCH) * n_seq
    nb = n_batch // BCH
    mesh = plsc.VectorSubcoreMesh(core_axis_name="core",
                                  subcore_axis_name="subcore")
    cparams = pltpu.CompilerParams(needs_layout_passes=False,
                                   use_tc_tiling_on_sc=True)

    @functools.partial(
        pl.kernel,
        out_type=jax.ShapeDtypeStruct((NW, L), jnp.float32),
        mesh=mesh,
        scratch_types=[
            pltpu.VMEM((BROW * C + 40,), jnp.float32),
            pltpu.VMEM((BROW * C + 40,), jnp.float32),
            pltpu.VMEM((NACC, L), jnp.float32),
        ],
        compiler_params=cparams,
    )
    def kern(yt_hbm, yp_hbm, b_hbm, out_hbm, bv, sv, accv):
        wid = lax.axis_index("subcore") * NC + lax.axis_index("core")
        pltpu.sync_copy(b_hbm, bv)
        zeros16 = jnp.zeros((L,), jnp.float32)
        for a in range(NACC):
            accv[a] = zeros16
        for k in range((BROW * C + 40) // L):
            sv[pl.ds(k * L, L)] = zeros16

        def body(yt_vmem, yp_vmem):
            @pl.loop(0, BCH // L)
            def _(g):
                l0 = g * L
                for r in range(SOCT):
                    t25 = yt_vmem[0, r, pl.ds(l0, L)] * BROW
                    for c in range(C):
                        p = yp_vmem[r, c, pl.ds(l0, L)]
                        plsc.addupdate_scatter(sv, [t25 + c], p)

        pltpu.emit_pipeline(
            body,
            grid=(grid,),
            in_specs=[
                pl.BlockSpec((1, SOCT, BCH),
                             lambda i: (i % n_seq, 0, i // n_seq)),
                pl.BlockSpec((SOCT, C, BCH),
                             lambda i: (i % n_seq, 0, i // n_seq)),
            ],
            out_specs=[],
            core_axis_name=("core", "subcore"),
            dimension_semantics=(pltpu.PARALLEL,),
        )(yt_hbm, yp_hbm)

        iota = lax.iota(jnp.int32, L)
        acc = jnp.zeros((L,), jnp.float32)
        for k in range(C * C // L):
            f = k * L
            flat = f + iota
            sval = plsc.load_gather(sv, [flat + flat // C])
            acc = acc + sval * bv[pl.ds(f, L)]
        accv[0] = acc
        pltpu.sync_copy(accv.at[0], out_hbm.at[wid])

    return kern(ytp, ypt, bpad)


def kernel(y_true, y_pred, B):
    seq = y_true.shape[1]
    ypt = jnp.transpose(y_pred, (1, 2, 0))
    ytp = jnp.transpose(y_true.astype(jnp.int32), (1, 0)).reshape(
        seq // SOCT, SOCT, -1)
    bpad = jnp.pad(B.reshape(-1), (0, BROW * C + 40 - C * C))
    partials = _sc_score(ytp, ypt, bpad)
    return jnp.sum(partials)


# speedup vs baseline: 1.3887x; 1.3887x over previous
import functools

import jax
import jax.numpy as jnp
from jax import lax
from jax.experimental import pallas as pl
from jax.experimental.pallas import tpu as pltpu
from jax.experimental.pallas import tpu_sc as plsc

C = 24
BROW = 25
L = 16
NC = 2
NS = 16
NW = NC * NS
BCH = 256
SOCT = 8
NACC = 4


def _sc_score(ytp, ypt, bpad):
    n_seq, _, n_batch = ytp.shape
    grid = (n_batch // BCH) * n_seq
    nb = n_batch // BCH
    mesh = plsc.VectorSubcoreMesh(core_axis_name="core",
                                  subcore_axis_name="subcore")
    cparams = pltpu.CompilerParams(needs_layout_passes=False,
                                   use_tc_tiling_on_sc=True)

    @functools.partial(
        pl.kernel,
        out_type=jax.ShapeDtypeStruct((NW, L), jnp.float32),
        mesh=mesh,
        scratch_types=[
            pltpu.VMEM((BROW * C + 40,), jnp.float32),
            pltpu.VMEM((L * C * C,), jnp.float32),
            pltpu.VMEM((NACC, L), jnp.float32),
        ],
        compiler_params=cparams,
    )
    def kern(yt_hbm, yp_hbm, b_hbm, out_hbm, bv, sv, accv):
        wid = lax.axis_index("subcore") * NC + lax.axis_index("core")
        pltpu.sync_copy(b_hbm, bv)
        zeros16 = jnp.zeros((L,), jnp.float32)
        for a in range(NACC):
            accv[a] = zeros16

        @pl.loop(0, L * C * C // L)
        def _(k):
            sv[pl.ds(k * L, L)] = zeros16

        iota0 = lax.iota(jnp.int32, L)

        def body(yt_vmem, yp_vmem):
            @pl.loop(0, BCH // L)
            def _(g):
                l0 = g * L
                for r in range(SOCT):
                    tb = yt_vmem[0, r, pl.ds(l0, L)] * L + iota0
                    for c in range(C):
                        p = yp_vmem[r, c, pl.ds(l0, L)]
                        plsc.addupdate_scatter(sv, [tb + (c * L * C)], p)

        pltpu.emit_pipeline(
            body,
            grid=(grid,),
            in_specs=[
                pl.BlockSpec((1, SOCT, BCH),
                             lambda i: (i % n_seq, 0, i // n_seq)),
                pl.BlockSpec((SOCT, C, BCH),
                             lambda i: (i % n_seq, 0, i // n_seq)),
            ],
            out_specs=[],
            core_axis_name=("core", "subcore"),
            dimension_semantics=(pltpu.PARALLEL,),
        )(yt_hbm, yp_hbm)

        iota16 = iota0 * L
        acc = jnp.zeros((L,), jnp.float32)
        for e0 in range(0, C * C, L):
            base = iota16 + e0 * L
            rsum = jnp.zeros((L,), jnp.float32)
            for k in range(L):
                rsum = rsum + plsc.load_gather(sv, [base + k])
            ent = e0 + iota0
            cc = ent // C
            tt = ent - cc * C
            bval = plsc.load_gather(bv, [tt * C + cc])
            acc = acc + rsum * bval
        accv[0] = acc
        pltpu.sync_copy(accv.at[0], out_hbm.at[wid])

    return kern(ytp, ypt, bpad)


def kernel(y_true, y_pred, B):
    seq = y_true.shape[1]
    ypt = jnp.transpose(y_pred, (1, 2, 0))
    ytp = jnp.transpose(y_true.astype(jnp.int32), (1, 0)).reshape(
        seq // SOCT, SOCT, -1)
    bpad = jnp.pad(B.reshape(-1), (0, BROW * C + 40 - C * C))
    partials = _sc_score(ytp, ypt, bpad)
    return jnp.sum(partials)


# speedup vs baseline: 5.1645x; 3.7190x over previous
import functools

import jax
import jax.numpy as jnp
from jax import lax
from jax.experimental import pallas as pl
from jax.experimental.pallas import tpu as pltpu
from jax.experimental.pallas import tpu_sc as plsc

C = 24
BROW = 25
L = 16
NC = 2
NS = 16
NW = NC * NS
BCH = 256
SOCT = 8
NACC = 4


def _sc_score(ytp, ypt, bpad, n_sc):
    n_seq = ytp.shape[0]
    grid = (n_sc // BCH) * n_seq
    mesh = plsc.VectorSubcoreMesh(core_axis_name="core",
                                  subcore_axis_name="subcore")
    cparams = pltpu.CompilerParams(needs_layout_passes=False,
                                   use_tc_tiling_on_sc=True)

    @functools.partial(
        pl.kernel,
        out_type=jax.ShapeDtypeStruct((NW, L), jnp.float32),
        mesh=mesh,
        scratch_types=[
            pltpu.VMEM((BROW * C + 40,), jnp.float32),
            pltpu.VMEM((NACC, L), jnp.float32),
        ],
        compiler_params=cparams,
    )
    def kern(yt_hbm, yp_hbm, b_hbm, out_hbm, bv, accv):
        wid = lax.axis_index("subcore") * NC + lax.axis_index("core")
        pltpu.sync_copy(b_hbm, bv)
        zeros16 = jnp.zeros((L,), jnp.float32)
        for a in range(NACC):
            accv[a] = zeros16

        def body(yt_vmem, yp_vmem):
            @pl.loop(0, BCH // L)
            def _(g):
                l0 = g * L
                for r in range(SOCT):
                    t25 = yt_vmem[0, r, pl.ds(l0, L)] * BROW
                    acc = [accv[a] for a in range(NACC)]
                    for c in range(C):
                        w = plsc.load_gather(bv, [t25 + c])
                        p = yp_vmem[r, c, pl.ds(l0, L)]
                        acc[c % NACC] = acc[c % NACC] + w * p
                    for a in range(NACC):
                        accv[a] = acc[a]

        pltpu.emit_pipeline(
            body,
            grid=(grid,),
            in_specs=[
                pl.BlockSpec((1, SOCT, BCH),
                             lambda i: (i % n_seq, 0, i // n_seq)),
                pl.BlockSpec((SOCT, C, BCH),
                             lambda i: (i % n_seq, 0, i // n_seq)),
            ],
            out_specs=[],
            core_axis_name=("core", "subcore"),
            dimension_semantics=(pltpu.PARALLEL,),
        )(yt_hbm, yp_hbm)

        acc = (accv[0] + accv[1]) + (accv[2] + accv[3])
        accv[0] = acc
        pltpu.sync_copy(accv.at[0], out_hbm.at[wid])

    return kern(ytp, ypt, bpad)


TCB = 2048


def _tc_score(ytp, ypt, B, n_sc):
    n_seq = ytp.shape[0]
    n_batch = ypt.shape[2]
    nchunk = (n_batch - n_sc) // TCB
    b0 = n_sc // TCB

    def body(t_ref, p_ref, b_ref, o_ref, a_ref):
        i = pl.program_id(0)
        j = pl.program_id(1)

        @pl.when(jnp.logical_and(i == 0, j == 0))
        def _():
            a_ref[...] = jnp.zeros((C, TCB), jnp.float32)

        bt = b_ref[...].T
        for r in range(SOCT):
            t = t_ref[0, r]
            oh = (lax.broadcasted_iota(jnp.int32, (C, TCB), 0)
                  == t[None, :]).astype(jnp.float32)
            w = jnp.dot(bt, oh, preferred_element_type=jnp.float32)
            a_ref[...] += w * p_ref[r]

        @pl.when(jnp.logical_and(i == nchunk - 1, j == n_seq - 1))
        def _():
            o_ref[0, 0] = jnp.sum(a_ref[...])

    return pl.pallas_call(
        body,
        grid=(nchunk, n_seq),
        in_specs=[
            pl.BlockSpec((1, SOCT, TCB), lambda i, j: (j, 0, i + b0)),
            pl.BlockSpec((SOCT, C, TCB), lambda i, j: (j, 0, i + b0)),
            pl.BlockSpec((C, C), lambda i, j: (0, 0)),
        ],
        out_specs=pl.BlockSpec(memory_space=pltpu.SMEM),
        out_shape=jax.ShapeDtypeStruct((1, 1), jnp.float32),
        scratch_shapes=[pltpu.VMEM((C, TCB), jnp.float32)],
    )(ytp, ypt, B)


def kernel(y_true, y_pred, B):
    seq = y_true.shape[1]
    n_sc = 6144
    ypt = jnp.transpose(y_pred, (1, 2, 0))
    ytp = jnp.transpose(y_true.astype(jnp.int32), (1, 0)).reshape(
        seq // SOCT, SOCT, -1)
    bpad = jnp.pad(B, ((0, 0), (0, BROW - C))).reshape(-1)
    bpad = jnp.pad(bpad, (0, 640 - BROW * C))
    partials = _sc_score(ytp, ypt, bpad, n_sc)
    tc_part = _tc_score(ytp, ypt, B, n_sc)
    return jnp.sum(partials) + tc_part[0, 0]
